# baseline (speedup 1.0000x reference)
import numpy as np
import jax
import jax.numpy as jnp
from jax.experimental import pallas as pl
from jax.experimental.pallas import tpu as pltpu

_B = 64
_V = 1_000_000
_BLK = 8192
_NC = (_V + _BLK - 1) // _BLK

_M32 = 0xFFFFFFFF


def _np_threefry2x32(k0, k1, x0, x1):
    rot1 = (13, 15, 26, 6)
    rot2 = (17, 29, 16, 24)
    ks = (k0, k1, k0 ^ k1 ^ 0x1BD11BDA)

    def rnd(v0, v1, r):
        v0 = (v0 + v1) & _M32
        v1 = ((v1 << r) | (v1 >> (32 - r))) & _M32
        return v0, v0 ^ v1

    x0 = (x0 + ks[0]) & _M32
    x1 = (x1 + ks[1]) & _M32
    for i, (ka, kb) in enumerate(
        ((ks[1], ks[2]), (ks[2], ks[0]), (ks[0], ks[1]),
         (ks[1], ks[2]), (ks[2], ks[0]))):
        rots = rot1 if i % 2 == 0 else rot2
        for r in rots:
            x0, x1 = rnd(x0, x1, r)
        x0 = (x0 + ka) & _M32
        x1 = (x1 + kb + i + 1) & _M32
    return x0, x1


_SK0, _SK1 = _np_threefry2x32(0, 42, 0, 0)
_SK2 = _SK0 ^ _SK1 ^ 0x1BD11BDA

_TINY = np.float32(np.finfo(np.float32).tiny)
_SPAN = np.float32(np.float32(1.0) - _TINY)


def _rotl(x, r):
    return (x << np.uint32(r)) | (x >> np.uint32(32 - r))


def _threefry_bits(j):
    rot1 = (13, 15, 26, 6)
    rot2 = (17, 29, 16, 24)
    ks = (np.uint32(_SK0), np.uint32(_SK1), np.uint32(_SK2))
    x0 = jnp.full(j.shape, ks[0], jnp.uint32)
    x1 = j + ks[1]
    for i, (ka, kb) in enumerate(
        ((ks[1], ks[2]), (ks[2], ks[0]), (ks[0], ks[1]),
         (ks[1], ks[2]), (ks[2], ks[0]))):
        rots = rot1 if i % 2 == 0 else rot2
        for r in rots:
            x0 = x0 + x1
            x1 = x0 ^ _rotl(x1, r)
        x0 = x0 + ka
        x1 = x1 + np.uint32((int(kb) + i + 1) & _M32)
    return x0 ^ x1


def _sample_kernel(w_ref, out_ref, sv_ref, si_ref, j0_ref, c0_ref):
    c = pl.program_id(0)

    @pl.when(c == 0)
    def _init():
        sv_ref[...] = jnp.full((_B, _BLK), -1.0, jnp.float32)
        si_ref[...] = jnp.zeros((_B, _BLK), jnp.int32)
        cl = jax.lax.broadcasted_iota(jnp.int32, (_B, _BLK), 1)
        rows = jax.lax.broadcasted_iota(jnp.int32, (_B, _BLK), 0)
        c0_ref[...] = cl
        j0_ref[...] = (rows * np.int32(_V) + cl).astype(jnp.uint32)

    base = c * _BLK
    cols = c0_ref[...] + base
    j = j0_ref[...] + base.astype(jnp.uint32)

    bits = _threefry_bits(j)
    fb = (bits >> np.uint32(9)) | np.uint32(0x3F800000)
    f = jax.lax.bitcast_convert_type(fb, jnp.float32) - np.float32(1.0)
    u = jnp.maximum(f, _TINY)
    e = -jnp.log(u)
    wp = jnp.maximum(w_ref[...], np.float32(1e-30))
    val = wp / e
    val = jnp.where(cols < _V, val, np.float32(-1.0))

    sv = sv_ref[...]
    take = val > sv
    sv_ref[...] = jnp.where(take, val, sv)
    si_ref[...] = jnp.where(take, cols, si_ref[...])

    @pl.when(c == _NC - 1)
    def _finish():
        v = sv_ref[...]
        idx = si_ref[...]
        mv = jnp.max(v, axis=1, keepdims=True)
        mi = jnp.min(jnp.where(v == mv, idx, np.int32(0x7FFFFFFF)),
                     axis=1, keepdims=True)
        out_ref[...] = mi


def kernel(input):
    out = pl.pallas_call(
        _sample_kernel,
        grid=(_NC,),
        in_specs=[pl.BlockSpec((_B, _BLK), lambda c: (0, c))],
        out_specs=pl.BlockSpec((_B, 1), lambda c: (0, 0)),
        out_shape=jax.ShapeDtypeStruct((_B, 1), jnp.int32),
        scratch_shapes=[
            pltpu.VMEM((_B, _BLK), jnp.float32),
            pltpu.VMEM((_B, _BLK), jnp.int32),
            pltpu.VMEM((_B, _BLK), jnp.uint32),
            pltpu.VMEM((_B, _BLK), jnp.int32),
        ],
        compiler_params=pltpu.CompilerParams(
            dimension_semantics=("arbitrary",),
        ),
    )(input)
    return out


# speedup vs baseline: 1.0179x; 1.0179x over previous
import numpy as np
import jax
import jax.numpy as jnp
from jax.experimental import pallas as pl
from jax.experimental.pallas import tpu as pltpu

_B = 64
_V = 1_000_000
_BLK = 4096
_NC = (_V + _BLK - 1) // _BLK

_M32 = 0xFFFFFFFF


def _np_threefry2x32(k0, k1, x0, x1):
    rot1 = (13, 15, 26, 6)
    rot2 = (17, 29, 16, 24)
    ks = (k0, k1, k0 ^ k1 ^ 0x1BD11BDA)

    def rnd(v0, v1, r):
        v0 = (v0 + v1) & _M32
        v1 = ((v1 << r) | (v1 >> (32 - r))) & _M32
        return v0, v0 ^ v1

    x0 = (x0 + ks[0]) & _M32
    x1 = (x1 + ks[1]) & _M32
    for i, (ka, kb) in enumerate(
        ((ks[1], ks[2]), (ks[2], ks[0]), (ks[0], ks[1]),
         (ks[1], ks[2]), (ks[2], ks[0]))):
        rots = rot1 if i % 2 == 0 else rot2
        for r in rots:
            x0, x1 = rnd(x0, x1, r)
        x0 = (x0 + ka) & _M32
        x1 = (x1 + kb + i + 1) & _M32
    return x0, x1


_SK0, _SK1 = _np_threefry2x32(0, 42, 0, 0)
_SK2 = _SK0 ^ _SK1 ^ 0x1BD11BDA

_TINY = np.float32(np.finfo(np.float32).tiny)
_SPAN = np.float32(np.float32(1.0) - _TINY)


def _rotl(x, r):
    return (x << np.uint32(r)) | (x >> np.uint32(32 - r))


def _threefry_bits(j):
    rot1 = (13, 15, 26, 6)
    rot2 = (17, 29, 16, 24)
    ks = (np.uint32(_SK0), np.uint32(_SK1), np.uint32(_SK2))
    x0 = jnp.full(j.shape, ks[0], jnp.uint32)
    x1 = j + ks[1]
    for i, (ka, kb) in enumerate(
        ((ks[1], ks[2]), (ks[2], ks[0]), (ks[0], ks[1]),
         (ks[1], ks[2]), (ks[2], ks[0]))):
        rots = rot1 if i % 2 == 0 else rot2
        for r in rots:
            x0 = x0 + x1
            x1 = x0 ^ _rotl(x1, r)
        x0 = x0 + ka
        x1 = x1 + np.uint32((int(kb) + i + 1) & _M32)
    return x0 ^ x1


def _sample_kernel(w_ref, out_ref, sv_ref, si_ref):
    c = pl.program_id(0)

    @pl.when(c == 0)
    def _init():
        sv_ref[...] = jnp.full((_B, _BLK), -1.0, jnp.float32)
        si_ref[...] = jnp.zeros((_B, _BLK), jnp.int32)

    cols = jax.lax.broadcasted_iota(jnp.int32, (_B, _BLK), 1) + c * _BLK
    rows = jax.lax.broadcasted_iota(jnp.int32, (_B, _BLK), 0)
    j = (rows * np.int32(_V) + cols).astype(jnp.uint32)

    bits = _threefry_bits(j)
    fb = (bits >> np.uint32(9)) | np.uint32(0x3F800000)
    f = jax.lax.bitcast_convert_type(fb, jnp.float32) - np.float32(1.0)
    u = jnp.maximum(f, _TINY)
    e = -jnp.log(u)
    wp = jnp.maximum(w_ref[...], np.float32(1e-30))
    val = wp / e
    val = jnp.where(cols < _V, val, np.float32(-1.0))

    sv = sv_ref[...]
    take = val > sv
    sv_ref[...] = jnp.where(take, val, sv)
    si_ref[...] = jnp.where(take, cols, si_ref[...])

    @pl.when(c == _NC - 1)
    def _finish():
        v = sv_ref[...]
        idx = si_ref[...]
        mv = jnp.max(v, axis=1, keepdims=True)
        mi = jnp.min(jnp.where(v == mv, idx, np.int32(0x7FFFFFFF)),
                     axis=1, keepdims=True)
        out_ref[...] = mi


def kernel(input):
    out = pl.pallas_call(
        _sample_kernel,
        grid=(_NC,),
        in_specs=[pl.BlockSpec((_B, _BLK), lambda c: (0, c))],
        out_specs=pl.BlockSpec((_B, 1), lambda c: (0, 0)),
        out_shape=jax.ShapeDtypeStruct((_B, 1), jnp.int32),
        scratch_shapes=[
            pltpu.VMEM((_B, _BLK), jnp.float32),
            pltpu.VMEM((_B, _BLK), jnp.int32),
        ],
        compiler_params=pltpu.CompilerParams(
            dimension_semantics=("arbitrary",),
        ),
    )(input)
    return out
